# baseline (speedup 1.0000x reference)
import functools

import jax
import jax.numpy as jnp
from jax import lax
from jax.experimental import pallas as pl
from jax.experimental.pallas import tpu as pltpu
from jax.experimental.pallas import tpu_sc as plsc

N_NODES = 100000
N_EDGES = 1600000
D_FEAT = 128
MEM_DIM = 100
TIME_DIM = 100
REL_DIM = 16
B = 4096
K = 20
ATT_DIM = 200
DH = 100
DHP = 128
N3 = 3 * B


_TROWS = 4000


def _table_body(nf_ref, mem_ref, wf_ref, mw_ref, out_ref):
    nf = nf_ref[...]
    mm = mem_ref[...] * mw_ref[0, 0]
    proj = jnp.dot(nf, wf_ref[...], preferred_element_type=jnp.float32)
    out_ref[...] = proj + jnp.pad(mm, ((0, 0), (0, 128 - MEM_DIM)))


def _build_table(node_features, memory, wf_pad, mw_arr):
    grid = (N_NODES // _TROWS,)
    return pl.pallas_call(
        _table_body,
        grid=grid,
        in_specs=[
            pl.BlockSpec((_TROWS, D_FEAT), lambda i: (i, 0)),
            pl.BlockSpec((_TROWS, MEM_DIM), lambda i: (i, 0)),
            pl.BlockSpec((D_FEAT, 128), lambda i: (0, 0)),
            pl.BlockSpec((1, 1), lambda i: (0, 0)),
        ],
        out_specs=pl.BlockSpec((_TROWS, 128), lambda i: (i, 0)),
        out_shape=jax.ShapeDtypeStruct((N_NODES, 128), jnp.float32),
    )(node_features, memory, wf_pad, mw_arr)


_ERB = 32000


def _epack_body(in_ref, out_ref):
    x3 = in_ref[...].reshape(_ERB // 8, 8, REL_DIM)
    for j in range(8):
        out_ref[:, pl.ds(j * REL_DIM, REL_DIM)] = x3[:, j, :]


def _pack_edges(edge_features):
    return pl.pallas_call(
        _epack_body,
        grid=(N_EDGES // _ERB,),
        in_specs=[pl.BlockSpec((_ERB, REL_DIM), lambda i: (i, 0))],
        out_specs=pl.BlockSpec((_ERB // 8, 128), lambda i: (i, 0)),
        out_shape=jax.ShapeDtypeStruct((N_EDGES // 8, 128), jnp.float32),
    )(edge_features)


_NW = 32
_CH = 128
_NBUF = 4
_NBR_PW = (N3 * K) // _NW
_NODE_PW = N3 // _NW
_NGRP = _NBR_PW // (_CH * _NBUF)
_BN = 128
_NB = B // _BN
_NCH = 4
_BQ = B // _NCH
_NBQ = _BQ // _BN


def _sc_gather(table, nodes_idx, nbr_idx):
    mesh = plsc.VectorSubcoreMesh(core_axis_name="c", subcore_axis_name="s")

    @functools.partial(
        pl.kernel,
        out_type=(
            jax.ShapeDtypeStruct((N3, 128), jnp.float32),
            jax.ShapeDtypeStruct((N3 * K, 128), jnp.float32),
        ),
        mesh=mesh,
        scratch_types=[
            pltpu.VMEM((_NODE_PW,), jnp.int32),
            pltpu.VMEM((_NBR_PW,), jnp.int32),
            pltpu.VMEM((_NBUF, _CH, 128), jnp.float32),
            pltpu.SemaphoreType.DMA,
        ],
    )
    def gather_kernel(table_hbm, nodes_hbm, nbr_hbm,
                      h0_out, nbr_out,
                      idxn_v, idxb_v, rows_v, sem_a):
        wid = lax.axis_index("s") * 2 + lax.axis_index("c")

        pltpu.sync_copy(nodes_hbm.at[pl.ds(wid * _NODE_PW, _NODE_PW)], idxn_v)
        pltpu.sync_copy(nbr_hbm.at[pl.ds(wid * _NBR_PW, _NBR_PW)], idxb_v)

        descs = []
        for b in range(_NODE_PW // _CH):
            descs.append(pltpu.async_copy(
                table_hbm.at[idxn_v.at[pl.ds(b * _CH, _CH)]],
                rows_v.at[b], sem_a))
        for d in descs:
            d.wait()
        for b in range(_NODE_PW // _CH):
            pltpu.sync_copy(rows_v.at[b],
                            h0_out.at[pl.ds(wid * _NODE_PW + b * _CH, _CH)])

        def group(g, _):
            base = g * (_NBUF * _CH)
            da = []
            for b in range(_NBUF):
                off = base + b * _CH
                da.append(pltpu.async_copy(
                    table_hbm.at[idxb_v.at[pl.ds(off, _CH)]],
                    rows_v.at[b], sem_a))
            for d in da:
                d.wait()
            for b in range(_NBUF):
                pltpu.sync_copy(
                    rows_v.at[b],
                    nbr_out.at[pl.ds(wid * _NBR_PW + base + b * _CH, _CH)])
            return 0

        lax.fori_loop(0, _NGRP, group, 0)

    return gather_kernel(table, nodes_idx, nbr_idx)


_NTILE = N_EDGES // 128
_TPW = 391
_TB = 8
_TGR = 49


def _sc_transpose_edges(eT):
    mesh = plsc.VectorSubcoreMesh(core_axis_name="c", subcore_axis_name="s")

    @functools.partial(
        pl.kernel,
        out_type=jax.ShapeDtypeStruct((N_EDGES // 8, 128), jnp.float32),
        mesh=mesh,
        compiler_params=pltpu.CompilerParams(needs_layout_passes=False),
        scratch_types=[
            pltpu.VMEM((_TB, 16, 128), jnp.float32),
            pltpu.VMEM((_TB, 16, 128), jnp.float32),
            pltpu.SemaphoreType.DMA,
            pltpu.SemaphoreType.DMA,
        ],
    )
    def tk(eT_hbm, out_hbm, tin, tout, sem_i, sem_o):
        wid = lax.axis_index("s") * 2 + lax.axis_index("c")
        base = jnp.minimum(wid * _TPW, _NTILE - _TPW)
        iota16 = lax.iota(jnp.int32, 16)

        def group(g, _):
            ts = [jnp.minimum(base + g * _TB + b, _NTILE - 1)
                  for b in range(_TB)]
            di = []
            for b in range(_TB):
                di.append(pltpu.async_copy(
                    eT_hbm.at[:, pl.ds(ts[b] * 128, 128)], tin.at[b], sem_i))
            for d in di:
                d.wait()
            do = []
            for b in range(_TB):
                for s2 in range(16):
                    for j in range(8):
                        vals = plsc.load_gather(
                            tin.at[b],
                            [iota16, jnp.full((16,), 8 * s2 + j, jnp.int32)])
                        tout[b, s2, pl.ds(j * 16, 16)] = vals
                do.append(pltpu.async_copy(
                    tout.at[b], out_hbm.at[pl.ds(ts[b] * 16, 16)], sem_o))
            for d in do:
                d.wait()
            return 0

        lax.fori_loop(0, _TGR, group, 0)

    return tk(eT)


_EPW = (3 * _BQ * K) // _NW
_EB = 5
_EGR = _EPW // (_CH * _EB)


def _sc_gather_edges(eidx, ef_rm):
    mesh = plsc.VectorSubcoreMesh(core_axis_name="c", subcore_axis_name="s")
    nrow = 3 * _BQ * K

    @functools.partial(
        pl.kernel,
        out_type=jax.ShapeDtypeStruct((nrow, REL_DIM), jnp.float32),
        mesh=mesh,
        compiler_params=pltpu.CompilerParams(needs_layout_passes=False),
        scratch_types=[
            pltpu.VMEM((_EPW,), jnp.int32),
            pltpu.VMEM((_EB, _CH, 128), jnp.float32),
            pltpu.VMEM((_EB, _CH), jnp.int32),
            pltpu.VMEM((_CH, REL_DIM), jnp.float32),
            pltpu.SemaphoreType.DMA,
        ],
    )
    def gather_kernel(eidx_hbm, ef_hbm, e_out,
                      idxe_v, rows_v, sidx_v, stage_v, sem_a):
        wid = lax.axis_index("s") * 2 + lax.axis_index("c")
        pltpu.sync_copy(eidx_hbm.at[pl.ds(wid * _EPW, _EPW)], idxe_v)
        iota16 = lax.iota(jnp.int32, 16)

        def egroup(g, _):
            base = g * (_EB * _CH)
            for b in range(_EB):
                for j in range(8):
                    ev = idxe_v[pl.ds(base + b * _CH + j * 16, 16)]
                    sidx_v[b, pl.ds(j * 16, 16)] = lax.shift_right_logical(ev, 3)
            db = []
            for b in range(_EB):
                db.append(pltpu.async_copy(
                    ef_hbm.at[sidx_v.at[b]], rows_v.at[b], sem_a))
            for d in db:
                d.wait()
            for b in range(_EB):
                for j in range(8):
                    ev = idxe_v[pl.ds(base + b * _CH + j * 16, 16)]
                    rj = iota16 + (j * 16)
                    cb = lax.shift_left(jnp.bitwise_and(ev, 7), 4)
                    for c in range(REL_DIM):
                        vals = plsc.load_gather(rows_v.at[b], [rj, cb + c])
                        plsc.store_scatter(
                            stage_v, [rj, jnp.full((16,), c, jnp.int32)], vals)
                pltpu.sync_copy(
                    stage_v,
                    e_out.at[pl.ds(wid * _EPW + base + b * _CH, _CH)])
            return 0

        lax.fori_loop(0, _EGR, egroup, 0)

    return gather_kernel(eidx, ef_rm)


_INV_PI = 0.31830988618379067
_PI_CHUNKS = (3.25, -0.109375, 0.0009765625, -7.6293945e-06,
              -1.1920929e-06, -8.940697e-08, 1.984187e-09)
_COS_COEF = (1.0, -0.5, 0.041666664, -0.0013888874, 2.480031e-05,
             -2.7499843e-07, 1.9591886e-09)


def _fast_cos(x):
    n = jnp.floor(x * jnp.float32(_INV_PI) + jnp.float32(0.5))
    r = x
    for c in _PI_CHUNKS:
        r = r - n * jnp.float32(c)
    u = r * r
    p = jnp.float32(_COS_COEF[6])
    for k in (5, 4, 3, 2, 1, 0):
        p = p * u + jnp.float32(_COS_COEF[k])
    ni = n.astype(jnp.int32)
    sign = jnp.where(jnp.bitwise_and(ni, 1) == 1,
                     jnp.float32(-1.0), jnp.float32(1.0))
    return sign * p


def _att_body(h0s_ref, h0d_ref, h0n_ref, nbs_ref, nbd_ref, nbn_ref,
              es_ref, ed_ref, en_ref, nts_ref, ntd_ref, ntn_ref, et_ref,
              wkn_ref, wke_ref, wkt_ref, wvn_ref, wve_ref, wvt_ref,
              wqa_ref, wqb_ref, woa_ref, wob_ref,
              m1a_ref, m1b_ref, mb1_ref, m2_ref, mb2_ref,
              a1a_ref, a1b_ref, ab1_ref, a2_ref, ab2_ref,
              tw_ref, tb_ref, pos_ref, neg_ref):
    bn = _BN
    tw = tw_ref[...]
    tb = tb_ref[...]
    tes = _fast_cos(tb)
    wkn, wke, wkt = wkn_ref[...], wke_ref[...], wkt_ref[...]
    wvn, wve, wvt = wvn_ref[...], wve_ref[...], wvt_ref[...]
    wqa, wqb = wqa_ref[...], wqb_ref[...]
    woa, wob = woa_ref[...], wob_ref[...]
    et = et_ref[...]

    def dot(a, b):
        return jnp.dot(a, b, preferred_element_type=jnp.float32)

    embs = []
    for h0_ref, nb_ref, e_ref, nt_ref in (
            (h0s_ref, nbs_ref, es_ref, nts_ref),
            (h0d_ref, nbd_ref, ed_ref, ntd_ref),
            (h0n_ref, nbn_ref, en_ref, ntn_ref)):
        nb = nb_ref[...].reshape(K * bn, 128)[:, :MEM_DIM]
        e = e_ref[...].reshape(K * bn, REL_DIM)
        nt = nt_ref[...]
        dt = (jnp.broadcast_to(et[None], (K, bn, 1)) - nt).reshape(K * bn, 1)
        te = _fast_cos(dt * tw + tb)
        kk = dot(nb, wkn) + dot(e, wke) + dot(te, wkt)
        vv = dot(nb, wvn) + dot(e, wve) + dot(te, wvt)
        h0 = h0_ref[...][:, :MEM_DIM]
        q = dot(h0, wqa) + dot(tes, wqb)
        outs = []
        for h in range(2):
            cs = slice(h * DHP, h * DHP + DHP)
            qh = q[:, cs]
            cols = [jnp.sum(qh * kk[k2 * bn:(k2 + 1) * bn, cs],
                            axis=1, keepdims=True) for k2 in range(K)]
            logits = jnp.concatenate(cols, axis=1) / 10.0
            mx = jnp.max(logits, axis=1, keepdims=True)
            p = jnp.exp(logits - mx)
            attn = p / jnp.sum(p, axis=1, keepdims=True)
            oh = attn[:, 0:1] * vv[0:bn, cs]
            for k2 in range(1, K):
                oh = oh + attn[:, k2:k2 + 1] * vv[k2 * bn:(k2 + 1) * bn, cs]
            outs.append(oh)
        att = dot(outs[0], woa) + dot(outs[1], wob)
        h1 = jnp.maximum(dot(att, m1a_ref[...]) + dot(h0, m1b_ref[...])
                         + mb1_ref[...], 0.0)
        embs.append(dot(h1, m2_ref[...]) + mb2_ref[...])

    emb_s, emb_d, emb_n = embs
    a1a, a1b, ab1 = a1a_ref[...], a1b_ref[...], ab1_ref[...]
    a2, ab2 = a2_ref[...], ab2_ref[...]
    zp = jnp.maximum(dot(emb_s, a1a) + dot(emb_d, a1b) + ab1, 0.0)
    pos_ref[...] = dot(zp, a2) + ab2
    zn = jnp.maximum(dot(emb_s, a1a) + dot(emb_n, a1b) + ab1, 0.0)
    neg_ref[...] = dot(zn, a2) + ab2


def _attention(h0g, nbr3, e3_c, nt3, et2, weights, c):
    def seg_spec(shape, seg):
        if len(shape) == 2:
            return pl.BlockSpec((_BN, shape[1]),
                                lambda i, s=seg: (s * _NB + c * _NBQ + i, 0))
        return pl.BlockSpec((K, _BN, shape[2]),
                            lambda i, s=seg: (0, s * _NB + c * _NBQ + i, 0))

    def eseg_spec(seg):
        return pl.BlockSpec((K, _BN, REL_DIM),
                            lambda i, s=seg: (0, s * _NBQ + i, 0))

    def full_spec(shape):
        return pl.BlockSpec(shape, lambda i: tuple(0 for _ in shape))

    in_specs = []
    args = []
    for seg in range(3):
        in_specs.append(seg_spec(h0g.shape, seg)); args.append(h0g)
    for seg in range(3):
        in_specs.append(seg_spec(nbr3.shape, seg)); args.append(nbr3)
    for seg in range(3):
        in_specs.append(eseg_spec(seg)); args.append(e3_c)
    for seg in range(3):
        in_specs.append(seg_spec(nt3.shape, seg)); args.append(nt3)
    in_specs.append(pl.BlockSpec((_BN, 1), lambda i: (c * _NBQ + i, 0)))
    args.append(et2)
    for w in weights:
        in_specs.append(full_spec(w.shape)); args.append(w)

    return pl.pallas_call(
        _att_body,
        grid=(_NBQ,),
        in_specs=in_specs,
        out_specs=[pl.BlockSpec((_BN, 64), lambda i: (i, 0)),
                   pl.BlockSpec((_BN, 64), lambda i: (i, 0))],
        out_shape=[jax.ShapeDtypeStruct((_BQ, 64), jnp.float32),
                   jax.ShapeDtypeStruct((_BQ, 64), jnp.float32)],
    )(*args)


def _pad_head_cols(w):
    r = w.shape[0]
    z = jnp.zeros((r, DHP - DH), jnp.float32)
    return jnp.concatenate([w[:, :DH], z, w[:, DH:], z], axis=1)


def kernel(source_nodes, destination_nodes, negative_nodes, edge_times,
           edge_idxs, neighbors, neighbor_edge_idxs, neighbor_times,
           memory, node_features, edge_features, time_w, time_b, W_feat,
           Wq, Wk, Wv, Wo, merge_w1, merge_b1, merge_w2, merge_b2,
           aff_w1, aff_b1, aff_w2, aff_b2, mem_weight, memEmb_weight):
    f32 = jnp.float32
    nodes = jnp.concatenate(
        [source_nodes, destination_nodes, negative_nodes]).astype(jnp.int32)
    nbr_flat = neighbors.T.reshape(-1).astype(jnp.int32)
    eidx_cks = (neighbor_edge_idxs.T.astype(jnp.int32)
                .reshape(K, 3, _NCH, _BQ).transpose(2, 0, 1, 3)
                .reshape(_NCH, 3 * _BQ * K))

    wf_pad = jnp.pad(W_feat.astype(f32) * memEmb_weight,
                     ((0, 0), (0, 128 - MEM_DIM)))
    mw_arr = jnp.reshape(mem_weight.astype(f32), (1, 1))
    table = _build_table(node_features, memory, wf_pad, mw_arr)

    ef_rm = _sc_transpose_edges(edge_features.T)
    h0g, nbrf = _sc_gather(table, nodes, nbr_flat)
    nbr3 = nbrf.reshape(K, N3, 128)
    nt3 = neighbor_times.T.reshape(K, N3, 1)
    et2 = edge_times.reshape(B, 1)

    weights = (
        _pad_head_cols(Wk[:MEM_DIM]),
        _pad_head_cols(Wk[MEM_DIM:MEM_DIM + REL_DIM]),
        _pad_head_cols(Wk[MEM_DIM + REL_DIM:]),
        _pad_head_cols(Wv[:MEM_DIM]),
        _pad_head_cols(Wv[MEM_DIM:MEM_DIM + REL_DIM]),
        _pad_head_cols(Wv[MEM_DIM + REL_DIM:]),
        _pad_head_cols(Wq[:MEM_DIM]),
        _pad_head_cols(Wq[MEM_DIM:]),
        jnp.pad(Wo[:DH], ((0, DHP - DH), (0, 0))),
        jnp.pad(Wo[DH:], ((0, DHP - DH), (0, 0))),
        merge_w1[:ATT_DIM],
        merge_w1[ATT_DIM:],
        merge_b1.reshape(1, MEM_DIM),
        merge_w2,
        merge_b2.reshape(1, MEM_DIM),
        aff_w1[:MEM_DIM],
        aff_w1[MEM_DIM:],
        aff_b1.reshape(1, MEM_DIM),
        aff_w2,
        aff_b2.reshape(1, 64),
        time_w.reshape(1, TIME_DIM),
        time_b.reshape(1, TIME_DIM),
    )
    pos_c, neg_c = [], []
    for c in range(_NCH):
        ef_c = _sc_gather_edges(eidx_cks[c], ef_rm)
        e3_c = ef_c.reshape(K, 3 * _BQ, REL_DIM)
        p_, n_ = _attention(h0g, nbr3, e3_c, nt3, et2, weights, c)
        pos_c.append(p_)
        neg_c.append(n_)
    return jnp.concatenate(pos_c), jnp.concatenate(neg_c)


# speedup vs baseline: 1.1305x; 1.1305x over previous
import functools

import jax
import jax.numpy as jnp
from jax import lax
from jax.experimental import pallas as pl
from jax.experimental.pallas import tpu as pltpu
from jax.experimental.pallas import tpu_sc as plsc

N_NODES = 100000
N_EDGES = 1600000
D_FEAT = 128
MEM_DIM = 100
TIME_DIM = 100
REL_DIM = 16
B = 4096
K = 20
ATT_DIM = 200
DH = 100
DHP = 128
N3 = 3 * B


_TROWS = 4000


def _table_body(nf_ref, mem_ref, wf_ref, mw_ref, out_ref):
    nf = nf_ref[...]
    mm = mem_ref[...] * mw_ref[0, 0]
    proj = jnp.dot(nf, wf_ref[...], preferred_element_type=jnp.float32)
    out_ref[...] = proj + jnp.pad(mm, ((0, 0), (0, 128 - MEM_DIM)))


def _build_table(node_features, memory, wf_pad, mw_arr):
    grid = (N_NODES // _TROWS,)
    return pl.pallas_call(
        _table_body,
        grid=grid,
        in_specs=[
            pl.BlockSpec((_TROWS, D_FEAT), lambda i: (i, 0)),
            pl.BlockSpec((_TROWS, MEM_DIM), lambda i: (i, 0)),
            pl.BlockSpec((D_FEAT, 128), lambda i: (0, 0)),
            pl.BlockSpec((1, 1), lambda i: (0, 0)),
        ],
        out_specs=pl.BlockSpec((_TROWS, 128), lambda i: (i, 0)),
        out_shape=jax.ShapeDtypeStruct((N_NODES, 128), jnp.float32),
    )(node_features, memory, wf_pad, mw_arr)


_ERB = 32000


def _epack_body(in_ref, out_ref):
    x3 = in_ref[...].reshape(_ERB // 8, 8, REL_DIM)
    for j in range(8):
        out_ref[:, pl.ds(j * REL_DIM, REL_DIM)] = x3[:, j, :]


def _pack_edges(edge_features):
    return pl.pallas_call(
        _epack_body,
        grid=(N_EDGES // _ERB,),
        in_specs=[pl.BlockSpec((_ERB, REL_DIM), lambda i: (i, 0))],
        out_specs=pl.BlockSpec((_ERB // 8, 128), lambda i: (i, 0)),
        out_shape=jax.ShapeDtypeStruct((N_EDGES // 8, 128), jnp.float32),
    )(edge_features)


_NW = 32
_CH = 128
_NBUF = 4
_NBR_PW = (N3 * K) // _NW
_NODE_PW = N3 // _NW
_NGRP = _NBR_PW // (_CH * _NBUF)
_BN = 128
_NB = B // _BN
_NCH = 4
_BQ = B // _NCH
_NBQ = _BQ // _BN


def _sc_gather(table, nodes_idx, nbr_idx):
    mesh = plsc.VectorSubcoreMesh(core_axis_name="c", subcore_axis_name="s")

    @functools.partial(
        pl.kernel,
        out_type=(
            jax.ShapeDtypeStruct((N3, 128), jnp.float32),
            jax.ShapeDtypeStruct((N3 * K, 128), jnp.float32),
        ),
        mesh=mesh,
        scratch_types=[
            pltpu.VMEM((_NODE_PW,), jnp.int32),
            pltpu.VMEM((_NBR_PW,), jnp.int32),
            pltpu.VMEM((_NBUF, _CH, 128), jnp.float32),
            pltpu.SemaphoreType.DMA,
        ],
    )
    def gather_kernel(table_hbm, nodes_hbm, nbr_hbm,
                      h0_out, nbr_out,
                      idxn_v, idxb_v, rows_v, sem_a):
        wid = lax.axis_index("s") * 2 + lax.axis_index("c")

        pltpu.sync_copy(nodes_hbm.at[pl.ds(wid * _NODE_PW, _NODE_PW)], idxn_v)
        pltpu.sync_copy(nbr_hbm.at[pl.ds(wid * _NBR_PW, _NBR_PW)], idxb_v)

        descs = []
        for b in range(_NODE_PW // _CH):
            descs.append(pltpu.async_copy(
                table_hbm.at[idxn_v.at[pl.ds(b * _CH, _CH)]],
                rows_v.at[b], sem_a))
        for d in descs:
            d.wait()
        for b in range(_NODE_PW // _CH):
            pltpu.sync_copy(rows_v.at[b],
                            h0_out.at[pl.ds(wid * _NODE_PW + b * _CH, _CH)])

        def group(g, _):
            base = g * (_NBUF * _CH)
            da = []
            for b in range(_NBUF):
                off = base + b * _CH
                da.append(pltpu.async_copy(
                    table_hbm.at[idxb_v.at[pl.ds(off, _CH)]],
                    rows_v.at[b], sem_a))
            for d in da:
                d.wait()
            for b in range(_NBUF):
                pltpu.sync_copy(
                    rows_v.at[b],
                    nbr_out.at[pl.ds(wid * _NBR_PW + base + b * _CH, _CH)])
            return 0

        lax.fori_loop(0, _NGRP, group, 0)

    return gather_kernel(table, nodes_idx, nbr_idx)


_EPW = (3 * _BQ * K) // _NW
_EB = 5
_EGR = _EPW // (_CH * _EB)


def _sc_gather_edges(eidx, ef_rm):
    mesh = plsc.VectorSubcoreMesh(core_axis_name="c", subcore_axis_name="s")
    nrow = 3 * _BQ * K

    @functools.partial(
        pl.kernel,
        out_type=jax.ShapeDtypeStruct((nrow, REL_DIM), jnp.float32),
        mesh=mesh,
        compiler_params=pltpu.CompilerParams(needs_layout_passes=False),
        scratch_types=[
            pltpu.VMEM((_EPW,), jnp.int32),
            pltpu.VMEM((_EB, _CH, 128), jnp.float32),
            pltpu.VMEM((_EB, _CH), jnp.int32),
            pltpu.VMEM((_CH, REL_DIM), jnp.float32),
            pltpu.SemaphoreType.DMA,
        ],
    )
    def gather_kernel(eidx_hbm, ef_hbm, e_out,
                      idxe_v, rows_v, sidx_v, stage_v, sem_a):
        wid = lax.axis_index("s") * 2 + lax.axis_index("c")
        pltpu.sync_copy(eidx_hbm.at[pl.ds(wid * _EPW, _EPW)], idxe_v)
        iota16 = lax.iota(jnp.int32, 16)

        def egroup(g, _):
            base = g * (_EB * _CH)
            for b in range(_EB):
                for j in range(8):
                    ev = idxe_v[pl.ds(base + b * _CH + j * 16, 16)]
                    sidx_v[b, pl.ds(j * 16, 16)] = lax.shift_right_logical(ev, 3)
            db = []
            for b in range(_EB):
                db.append(pltpu.async_copy(
                    ef_hbm.at[sidx_v.at[b]], rows_v.at[b], sem_a))
            for d in db:
                d.wait()
            for b in range(_EB):
                for j in range(8):
                    ev = idxe_v[pl.ds(base + b * _CH + j * 16, 16)]
                    rj = iota16 + (j * 16)
                    cb = lax.shift_left(jnp.bitwise_and(ev, 7), 4)
                    for c in range(REL_DIM):
                        vals = plsc.load_gather(rows_v.at[b], [rj, cb + c])
                        plsc.store_scatter(
                            stage_v, [rj, jnp.full((16,), c, jnp.int32)], vals)
                pltpu.sync_copy(
                    stage_v,
                    e_out.at[pl.ds(wid * _EPW + base + b * _CH, _CH)])
            return 0

        lax.fori_loop(0, _EGR, egroup, 0)

    return gather_kernel(eidx, ef_rm)


_INV_PI = 0.31830988618379067
_PI_CHUNKS = (3.25, -0.109375, 0.0009765625, -7.6293945e-06,
              -1.1920929e-06, -8.940697e-08, 1.984187e-09)
_COS_COEF = (1.0, -0.5, 0.041666664, -0.0013888874, 2.480031e-05,
             -2.7499843e-07, 1.9591886e-09)


def _fast_cos(x):
    n = jnp.floor(x * jnp.float32(_INV_PI) + jnp.float32(0.5))
    r = x
    for c in _PI_CHUNKS:
        r = r - n * jnp.float32(c)
    u = r * r
    p = jnp.float32(_COS_COEF[6])
    for k in (5, 4, 3, 2, 1, 0):
        p = p * u + jnp.float32(_COS_COEF[k])
    ni = n.astype(jnp.int32)
    sign = jnp.where(jnp.bitwise_and(ni, 1) == 1,
                     jnp.float32(-1.0), jnp.float32(1.0))
    return sign * p


def _att_body(h0s_ref, h0d_ref, h0n_ref, nbs_ref, nbd_ref, nbn_ref,
              es_ref, ed_ref, en_ref, nts_ref, ntd_ref, ntn_ref, et_ref,
              wkn_ref, wke_ref, wkt_ref, wvn_ref, wve_ref, wvt_ref,
              wqa_ref, wqb_ref, woa_ref, wob_ref,
              m1a_ref, m1b_ref, mb1_ref, m2_ref, mb2_ref,
              a1a_ref, a1b_ref, ab1_ref, a2_ref, ab2_ref,
              tw_ref, tb_ref, pos_ref, neg_ref):
    bn = _BN
    tw = tw_ref[...]
    tb = tb_ref[...]
    tes = _fast_cos(tb)
    wkn, wke, wkt = wkn_ref[...], wke_ref[...], wkt_ref[...]
    wvn, wve, wvt = wvn_ref[...], wve_ref[...], wvt_ref[...]
    wqa, wqb = wqa_ref[...], wqb_ref[...]
    woa, wob = woa_ref[...], wob_ref[...]
    et = et_ref[...]

    def dot(a, b):
        return jnp.dot(a, b, preferred_element_type=jnp.float32)

    embs = []
    for h0_ref, nb_ref, e_ref, nt_ref in (
            (h0s_ref, nbs_ref, es_ref, nts_ref),
            (h0d_ref, nbd_ref, ed_ref, ntd_ref),
            (h0n_ref, nbn_ref, en_ref, ntn_ref)):
        nb = nb_ref[...].reshape(K * bn, 128)[:, :MEM_DIM]
        e = e_ref[...].reshape(K * bn, REL_DIM)
        nt = nt_ref[...]
        dt = (jnp.broadcast_to(et[None], (K, bn, 1)) - nt).reshape(K * bn, 1)
        te = _fast_cos(dt * tw + tb)
        kk = dot(nb, wkn) + dot(e, wke) + dot(te, wkt)
        vv = dot(nb, wvn) + dot(e, wve) + dot(te, wvt)
        h0 = h0_ref[...][:, :MEM_DIM]
        q = dot(h0, wqa) + dot(tes, wqb)
        outs = []
        for h in range(2):
            cs = slice(h * DHP, h * DHP + DHP)
            qh = q[:, cs]
            cols = [jnp.sum(qh * kk[k2 * bn:(k2 + 1) * bn, cs],
                            axis=1, keepdims=True) for k2 in range(K)]
            logits = jnp.concatenate(cols, axis=1) / 10.0
            mx = jnp.max(logits, axis=1, keepdims=True)
            p = jnp.exp(logits - mx)
            attn = p / jnp.sum(p, axis=1, keepdims=True)
            oh = attn[:, 0:1] * vv[0:bn, cs]
            for k2 in range(1, K):
                oh = oh + attn[:, k2:k2 + 1] * vv[k2 * bn:(k2 + 1) * bn, cs]
            outs.append(oh)
        att = dot(outs[0], woa) + dot(outs[1], wob)
        h1 = jnp.maximum(dot(att, m1a_ref[...]) + dot(h0, m1b_ref[...])
                         + mb1_ref[...], 0.0)
        embs.append(dot(h1, m2_ref[...]) + mb2_ref[...])

    emb_s, emb_d, emb_n = embs
    a1a, a1b, ab1 = a1a_ref[...], a1b_ref[...], ab1_ref[...]
    a2, ab2 = a2_ref[...], ab2_ref[...]
    zp = jnp.maximum(dot(emb_s, a1a) + dot(emb_d, a1b) + ab1, 0.0)
    pos_ref[...] = dot(zp, a2) + ab2
    zn = jnp.maximum(dot(emb_s, a1a) + dot(emb_n, a1b) + ab1, 0.0)
    neg_ref[...] = dot(zn, a2) + ab2


def _attention(h0g, nbr3, e3_c, nt3, et2, weights, c):
    def seg_spec(shape, seg):
        if len(shape) == 2:
            return pl.BlockSpec((_BN, shape[1]),
                                lambda i, s=seg: (s * _NB + c * _NBQ + i, 0))
        return pl.BlockSpec((K, _BN, shape[2]),
                            lambda i, s=seg: (0, s * _NB + c * _NBQ + i, 0))

    def eseg_spec(seg):
        return pl.BlockSpec((K, _BN, REL_DIM),
                            lambda i, s=seg: (0, s * _NBQ + i, 0))

    def full_spec(shape):
        return pl.BlockSpec(shape, lambda i: tuple(0 for _ in shape))

    in_specs = []
    args = []
    for seg in range(3):
        in_specs.append(seg_spec(h0g.shape, seg)); args.append(h0g)
    for seg in range(3):
        in_specs.append(seg_spec(nbr3.shape, seg)); args.append(nbr3)
    for seg in range(3):
        in_specs.append(eseg_spec(seg)); args.append(e3_c)
    for seg in range(3):
        in_specs.append(seg_spec(nt3.shape, seg)); args.append(nt3)
    in_specs.append(pl.BlockSpec((_BN, 1), lambda i: (c * _NBQ + i, 0)))
    args.append(et2)
    for w in weights:
        in_specs.append(full_spec(w.shape)); args.append(w)

    return pl.pallas_call(
        _att_body,
        grid=(_NBQ,),
        in_specs=in_specs,
        out_specs=[pl.BlockSpec((_BN, 64), lambda i: (i, 0)),
                   pl.BlockSpec((_BN, 64), lambda i: (i, 0))],
        out_shape=[jax.ShapeDtypeStruct((_BQ, 64), jnp.float32),
                   jax.ShapeDtypeStruct((_BQ, 64), jnp.float32)],
    )(*args)


def _pad_head_cols(w):
    r = w.shape[0]
    z = jnp.zeros((r, DHP - DH), jnp.float32)
    return jnp.concatenate([w[:, :DH], z, w[:, DH:], z], axis=1)


def kernel(source_nodes, destination_nodes, negative_nodes, edge_times,
           edge_idxs, neighbors, neighbor_edge_idxs, neighbor_times,
           memory, node_features, edge_features, time_w, time_b, W_feat,
           Wq, Wk, Wv, Wo, merge_w1, merge_b1, merge_w2, merge_b2,
           aff_w1, aff_b1, aff_w2, aff_b2, mem_weight, memEmb_weight):
    f32 = jnp.float32
    nodes = jnp.concatenate(
        [source_nodes, destination_nodes, negative_nodes]).astype(jnp.int32)
    nbr_flat = neighbors.T.reshape(-1).astype(jnp.int32)
    eidx_cks = (neighbor_edge_idxs.T.astype(jnp.int32)
                .reshape(K, 3, _NCH, _BQ).transpose(2, 0, 1, 3)
                .reshape(_NCH, 3 * _BQ * K))

    wf_pad = jnp.pad(W_feat.astype(f32) * memEmb_weight,
                     ((0, 0), (0, 128 - MEM_DIM)))
    mw_arr = jnp.reshape(mem_weight.astype(f32), (1, 1))
    table = _build_table(node_features, memory, wf_pad, mw_arr)

    ef_rm = edge_features.reshape(N_EDGES // 8, 128)
    h0g, nbrf = _sc_gather(table, nodes, nbr_flat)
    nbr3 = nbrf.reshape(K, N3, 128)
    nt3 = neighbor_times.T.reshape(K, N3, 1)
    et2 = edge_times.reshape(B, 1)

    weights = (
        _pad_head_cols(Wk[:MEM_DIM]),
        _pad_head_cols(Wk[MEM_DIM:MEM_DIM + REL_DIM]),
        _pad_head_cols(Wk[MEM_DIM + REL_DIM:]),
        _pad_head_cols(Wv[:MEM_DIM]),
        _pad_head_cols(Wv[MEM_DIM:MEM_DIM + REL_DIM]),
        _pad_head_cols(Wv[MEM_DIM + REL_DIM:]),
        _pad_head_cols(Wq[:MEM_DIM]),
        _pad_head_cols(Wq[MEM_DIM:]),
        jnp.pad(Wo[:DH], ((0, DHP - DH), (0, 0))),
        jnp.pad(Wo[DH:], ((0, DHP - DH), (0, 0))),
        merge_w1[:ATT_DIM],
        merge_w1[ATT_DIM:],
        merge_b1.reshape(1, MEM_DIM),
        merge_w2,
        merge_b2.reshape(1, MEM_DIM),
        aff_w1[:MEM_DIM],
        aff_w1[MEM_DIM:],
        aff_b1.reshape(1, MEM_DIM),
        aff_w2,
        aff_b2.reshape(1, 64),
        time_w.reshape(1, TIME_DIM),
        time_b.reshape(1, TIME_DIM),
    )
    pos_c, neg_c = [], []
    for c in range(_NCH):
        ef_c = _sc_gather_edges(eidx_cks[c], ef_rm)
        e3_c = ef_c.reshape(K, 3 * _BQ, REL_DIM)
        p_, n_ = _attention(h0g, nbr3, e3_c, nt3, et2, weights, c)
        pos_c.append(p_)
        neg_c.append(n_)
    return jnp.concatenate(pos_c), jnp.concatenate(neg_c)
